# Initial kernel scaffold; baseline (speedup 1.0000x reference)
import jax
import jax.numpy as jnp
from jax.experimental import pallas as pl


def kernel(x, ln_w, ln_b, w_shared_in, w_shared_out, b_shared_out, w_expert_in, expert_out_w, expert_out_b, w_group_gate, w_expert_gate, group_bias, expert_bias):
    raise NotImplementedError("write your pallas kernel here")


# speedup vs baseline: 3.5316x; 3.5316x over previous
import functools

import jax
import jax.numpy as jnp
from jax.experimental import pallas as pl
from jax.experimental.pallas import tpu as pltpu

_EPS = 1e-5
_TOPK = 2
_BLK = 256


def _dot_t(a, b, precision=None):
    return jax.lax.dot_general(
        a, b, dimension_numbers=(((1,), (1,)), ((), ())),
        precision=precision, preferred_element_type=jnp.float32)


def _silu(v):
    return v * jax.nn.sigmoid(v)


def _fused(G, EPG, H, x_ref, lnw_ref, lnb_ref, wsi_ref, wso_ref, bso_ref,
           wei_ref, weo_ref, beo_ref, wg_ref, we_ref, gb_ref, eb_ref, out_ref):
    E = G * EPG
    xb = x_ref[...]
    mu = jnp.mean(xb, axis=-1, keepdims=True)
    xc = xb - mu
    var = jnp.mean(xc * xc, axis=-1, keepdims=True)
    flat = xc / jnp.sqrt(var + _EPS) * lnw_ref[...] + lnb_ref[...]

    hs = _dot_t(flat, wsi_ref[...])
    h_shared = _silu(hs[:, :H]) * hs[:, H:]
    acc = _dot_t(h_shared, wso_ref[...]) + bso_ref[...]

    he = _dot_t(flat, wei_ref[...])
    h_expert = _silu(he[:, :H]) * he[:, H:]

    g = _dot_t(flat, wg_ref[...]) + gb_ref[...]
    el = _dot_t(flat, we_ref[...])
    eb = eb_ref[...]

    gmask = g[:, 1:2] > g[:, 0:1]
    e4 = (jnp.where(gmask, el[:, EPG:], el[:, :EPG])
          + jnp.where(gmask, eb[:, EPG:], eb[:, :EPG]))
    m = jnp.max(e4, axis=-1, keepdims=True)
    ex = jnp.exp(e4 - m)
    p = ex / jnp.sum(ex, axis=-1, keepdims=True)

    cols = [p[:, i:i + 1] for i in range(EPG)]
    keep = []
    for e in range(EPG):
        cnt = jnp.zeros_like(cols[0])
        for j in range(EPG):
            if j == e:
                continue
            beats = (cols[j] >= cols[e]) if j < e else (cols[j] > cols[e])
            cnt = cnt + beats.astype(jnp.float32)
        keep.append((cnt < float(_TOPK)).astype(jnp.float32))
    c4 = p * jnp.concatenate(keep, axis=-1)
    gm = gmask.astype(jnp.float32)
    c8 = jnp.concatenate([c4 * (1.0 - gm), c4 * gm], axis=-1)

    for e in range(E):
        w = c8[:, e:e + 1]
        acc = acc + w * (_dot_t(h_expert, weo_ref[e]) + beo_ref[e:e + 1, :])
    out_ref[...] = acc


def kernel(x, ln_w, ln_b, w_shared_in, w_shared_out, b_shared_out,
           w_expert_in, expert_out_w, expert_out_b,
           w_group_gate, w_expert_gate, group_bias, expert_bias):
    B, T, C = x.shape
    S = B * T
    G = w_group_gate.shape[0]
    E = expert_out_w.shape[0]
    EPG = E // G
    H = w_shared_out.shape[1]
    flat_x = x.reshape(S, C)

    const2 = lambda i: (0, 0)
    const3 = lambda i: (0, 0, 0)
    out = pl.pallas_call(
        functools.partial(_fused, G, EPG, H),
        grid=(S // _BLK,),
        in_specs=[
            pl.BlockSpec((_BLK, C), lambda i: (i, 0)),
            pl.BlockSpec((1, C), const2),
            pl.BlockSpec((1, C), const2),
            pl.BlockSpec((2 * H, C), const2),
            pl.BlockSpec((C, H), const2),
            pl.BlockSpec((1, C), const2),
            pl.BlockSpec((2 * H, C), const2),
            pl.BlockSpec((E, C, H), const3),
            pl.BlockSpec((E, C), const2),
            pl.BlockSpec((G, C), const2),
            pl.BlockSpec((E, C), const2),
            pl.BlockSpec((1, G), const2),
            pl.BlockSpec((1, E), const2),
        ],
        out_specs=pl.BlockSpec((_BLK, C), lambda i: (i, 0)),
        out_shape=jax.ShapeDtypeStruct((S, C), jnp.float32),
        compiler_params=pltpu.CompilerParams(
            dimension_semantics=("parallel",),
            vmem_limit_bytes=128 * 1024 * 1024,
        ),
    )(flat_x, ln_w.reshape(1, C), ln_b.reshape(1, C), w_shared_in,
      w_shared_out, b_shared_out.reshape(1, C), w_expert_in, expert_out_w,
      expert_out_b, w_group_gate, w_expert_gate,
      group_bias.reshape(1, G), expert_bias.reshape(1, E))
    return out.reshape(B, T, C)


# speedup vs baseline: 3.6125x; 1.0229x over previous
import functools

import jax
import jax.numpy as jnp
from jax.experimental import pallas as pl
from jax.experimental.pallas import tpu as pltpu

_EPS = 1e-5
_TOPK = 2
_BLK = 512


def _dot_t(a, b, precision=None):
    return jax.lax.dot_general(
        a, b, dimension_numbers=(((1,), (1,)), ((), ())),
        precision=precision, preferred_element_type=jnp.float32)


def _silu(v):
    return v * jax.nn.sigmoid(v)


def _fused(G, EPG, H, x_ref, lnw_ref, lnb_ref, wsi_ref, wso_ref, bso_ref,
           wei_ref, weo_ref, beo_ref, wg_ref, we_ref, gb_ref, eb_ref, out_ref):
    E = G * EPG
    xb = x_ref[...]
    mu = jnp.mean(xb, axis=-1, keepdims=True)
    xc = xb - mu
    var = jnp.mean(xc * xc, axis=-1, keepdims=True)
    flat = xc / jnp.sqrt(var + _EPS) * lnw_ref[...] + lnb_ref[...]

    h_shared = (_silu(_dot_t(flat, wsi_ref[:H, :]))
                * _dot_t(flat, wsi_ref[H:, :]))
    acc = _dot_t(h_shared, wso_ref[...]) + bso_ref[...]

    h_expert = (_silu(_dot_t(flat, wei_ref[:H, :]))
                * _dot_t(flat, wei_ref[H:, :]))

    g = _dot_t(flat, wg_ref[...]) + gb_ref[...]
    el = _dot_t(flat, we_ref[...])
    eb = eb_ref[...]

    gmask = g[:, 1:2] > g[:, 0:1]
    e4 = (jnp.where(gmask, el[:, EPG:], el[:, :EPG])
          + jnp.where(gmask, eb[:, EPG:], eb[:, :EPG]))
    m = jnp.max(e4, axis=-1, keepdims=True)
    ex = jnp.exp(e4 - m)
    p = ex / jnp.sum(ex, axis=-1, keepdims=True)

    cols = [p[:, i:i + 1] for i in range(EPG)]
    keep = []
    for e in range(EPG):
        cnt = jnp.zeros_like(cols[0])
        for j in range(EPG):
            if j == e:
                continue
            beats = (cols[j] >= cols[e]) if j < e else (cols[j] > cols[e])
            cnt = cnt + beats.astype(jnp.float32)
        keep.append((cnt < float(_TOPK)).astype(jnp.float32))
    c4 = p * jnp.concatenate(keep, axis=-1)
    gm = gmask.astype(jnp.float32)
    c8 = jnp.concatenate([c4 * (1.0 - gm), c4 * gm], axis=-1)

    for e in range(E):
        w = c8[:, e:e + 1]
        acc = acc + w * (_dot_t(h_expert, weo_ref[e]) + beo_ref[e:e + 1, :])
    out_ref[...] = acc


def kernel(x, ln_w, ln_b, w_shared_in, w_shared_out, b_shared_out,
           w_expert_in, expert_out_w, expert_out_b,
           w_group_gate, w_expert_gate, group_bias, expert_bias):
    B, T, C = x.shape
    S = B * T
    G = w_group_gate.shape[0]
    E = expert_out_w.shape[0]
    EPG = E // G
    H = w_shared_out.shape[1]
    flat_x = x.reshape(S, C)

    const2 = lambda i: (0, 0)
    const3 = lambda i: (0, 0, 0)
    out = pl.pallas_call(
        functools.partial(_fused, G, EPG, H),
        grid=(S // _BLK,),
        in_specs=[
            pl.BlockSpec((_BLK, C), lambda i: (i, 0)),
            pl.BlockSpec((1, C), const2),
            pl.BlockSpec((1, C), const2),
            pl.BlockSpec((2 * H, C), const2),
            pl.BlockSpec((C, H), const2),
            pl.BlockSpec((1, C), const2),
            pl.BlockSpec((2 * H, C), const2),
            pl.BlockSpec((E, C, H), const3),
            pl.BlockSpec((E, C), const2),
            pl.BlockSpec((G, C), const2),
            pl.BlockSpec((E, C), const2),
            pl.BlockSpec((1, G), const2),
            pl.BlockSpec((1, E), const2),
        ],
        out_specs=pl.BlockSpec((_BLK, C), lambda i: (i, 0)),
        out_shape=jax.ShapeDtypeStruct((S, C), jnp.float32),
        compiler_params=pltpu.CompilerParams(
            dimension_semantics=("parallel",),
            vmem_limit_bytes=128 * 1024 * 1024,
        ),
    )(flat_x, ln_w.reshape(1, C), ln_b.reshape(1, C), w_shared_in,
      w_shared_out, b_shared_out.reshape(1, C), w_expert_in, expert_out_w,
      expert_out_b, w_group_gate, w_expert_gate,
      group_bias.reshape(1, G), expert_bias.reshape(1, E))
    return out.reshape(B, T, C)
